# baseline (speedup 1.0000x reference)
import functools

import jax
import jax.numpy as jnp
from jax import lax
from jax.experimental import pallas as pl
from jax.experimental.pallas import tpu as pltpu
from jax.experimental.pallas import tpu_sc as plsc

N = 10000
E = 320000
H = 8
OUT = 16
NC = 2
NS = 16
LANES = 16
N_PAD = 10240
E_PER_W = E // (NC * NS)
CHUNK = 128
CH = -(-E_PER_W // CHUNK)
E_PAD_W = CH * CHUNK


def _sc_degree_kernel():
    mesh = plsc.VectorSubcoreMesh(core_axis_name="c", subcore_axis_name="s")

    @functools.partial(
        pl.kernel,
        mesh=mesh,
        out_type=jax.ShapeDtypeStruct((NC, N_PAD), jnp.float32),
        scratch_types=[
            pltpu.VMEM((CH, CHUNK), jnp.int32),
            pltpu.VMEM((CHUNK,), jnp.float32),
            pltpu.VMEM_SHARED((N_PAD,), jnp.float32),
        ],
    )
    def sc_deg(dst_hbm, zeros_hbm, out_hbm, idx_v, ones_v, deg_sh):
        c = lax.axis_index("c")
        s = lax.axis_index("s")

        for i in range(CHUNK // LANES):
            ones_v[pl.ds(i * LANES, LANES)] = jnp.full(
                (LANES,), 1.0, jnp.float32)

        @pl.when(s == 0)
        def _():
            pltpu.sync_copy(zeros_hbm, deg_sh)

        plsc.subcore_barrier()

        pltpu.sync_copy(dst_hbm.at[c, s], idx_v)

        def body(j, carry):
            pltpu.sync_copy(ones_v, deg_sh.at[idx_v.at[j]], add=True)
            return carry

        lax.fori_loop(0, CH, body, 0)

        plsc.subcore_barrier()

        @pl.when(s == 0)
        def _():
            pltpu.sync_copy(deg_sh, out_hbm.at[c])

    return sc_deg


def _tc_body(ft_ref, deg_ref, bias_ref, out_ref):
    d = deg_ref[...]
    mask = (d[:, 0:1] + d[:, 1:2]) > 0.0
    acc = jnp.sum(ft_ref[...], axis=1)
    bias_mean = jnp.mean(bias_ref[...], axis=0, keepdims=True)
    out_ref[...] = jnp.where(mask, acc * (1.0 / H), 0.0) + bias_mean


def kernel(ft, e_ft, edge_index, W, bias):
    del e_ft, W
    n, h, out = ft.shape

    dst = edge_index[1]
    dst_pad = jnp.concatenate(
        [dst, jnp.full((NC * NS * E_PAD_W - E,), N, jnp.int32)]
    ).reshape(NC, NS, CH, CHUNK)
    zeros = jnp.zeros((N_PAD,), jnp.float32)

    deg2 = _sc_degree_kernel()(dst_pad, zeros)
    deg_t = jnp.swapaxes(deg2, 0, 1)[:n]

    bias2 = bias.reshape(h, out)

    return pl.pallas_call(
        _tc_body,
        out_shape=jax.ShapeDtypeStruct((n, out), jnp.float32),
    )(ft, deg_t, bias2)


# speedup vs baseline: 1.4456x; 1.4456x over previous
import functools

import jax
import jax.numpy as jnp
from jax import lax
from jax.experimental import pallas as pl
from jax.experimental.pallas import tpu as pltpu
from jax.experimental.pallas import tpu_sc as plsc

N = 10000
E = 320000
H = 8
OUT = 16
NC = 2
NS = 16
LANES = 16
N_PAD = 10240
E_PER_W = E // (NC * NS)
CHUNK = 128
CH = -(-E_PER_W // CHUNK)
E_PAD_W = CH * CHUNK


def _sc_degree_kernel():
    mesh = plsc.VectorSubcoreMesh(core_axis_name="c", subcore_axis_name="s")

    @functools.partial(
        pl.kernel,
        mesh=mesh,
        out_type=jax.ShapeDtypeStruct((NC, N_PAD), jnp.float32),
        scratch_types=[
            pltpu.VMEM((CH, CHUNK), jnp.int32),
            pltpu.VMEM((CHUNK,), jnp.float32),
            pltpu.VMEM_SHARED((N_PAD,), jnp.float32),
        ],
    )
    def sc_deg(dst_hbm, zeros_hbm, out_hbm, idx_v, ones_v, deg_sh):
        c = lax.axis_index("c")
        s = lax.axis_index("s")

        for i in range(CHUNK // LANES):
            ones_v[pl.ds(i * LANES, LANES)] = jnp.full(
                (LANES,), 1.0, jnp.float32)

        @pl.when(s == 0)
        def _():
            pltpu.sync_copy(zeros_hbm, deg_sh)

        plsc.subcore_barrier()

        pltpu.sync_copy(dst_hbm.at[c, s], idx_v)

        def body(j, carry):
            pltpu.sync_copy(ones_v, deg_sh.at[idx_v.at[j]], add=True)
            return carry

        lax.fori_loop(0, CH, body, 0)

        plsc.subcore_barrier()

        @pl.when(s == 0)
        def _():
            pltpu.sync_copy(deg_sh, out_hbm.at[c])

    return sc_deg


def _tc_body(ft_ref, deg_ref, bias_ref, out_ref):
    d = deg_ref[...]
    mask = (d[:, 0:1] + d[:, 1:2]) > 0.0
    x = ft_ref[...]
    row = lax.broadcasted_iota(jnp.int32, (H * OUT, OUT), 0)
    col = lax.broadcasted_iota(jnp.int32, (H * OUT, OUT), 1)
    s = jnp.where(row % OUT == col, 1.0 / H, 0.0)
    acc = jnp.dot(x, s, preferred_element_type=jnp.float32)
    bias_mean = jnp.mean(bias_ref[...], axis=0, keepdims=True)
    out_ref[...] = jnp.where(mask, acc, 0.0) + bias_mean


def kernel(ft, e_ft, edge_index, W, bias):
    del e_ft, W
    n, h, out = ft.shape

    dst = edge_index[1]
    dst_pad = jnp.concatenate(
        [dst, jnp.full((NC * NS * E_PAD_W - E,), N, jnp.int32)]
    ).reshape(NC, NS, CH, CHUNK)
    zeros = jnp.zeros((N_PAD,), jnp.float32)

    deg2 = _sc_degree_kernel()(dst_pad, zeros)
    deg_t = jnp.swapaxes(deg2, 0, 1)[:n]

    bias2 = bias.reshape(h, out)

    return pl.pallas_call(
        _tc_body,
        out_shape=jax.ShapeDtypeStruct((n, out), jnp.float32),
    )(ft.reshape(n, h * out), deg_t, bias2)
